# Initial kernel scaffold; baseline (speedup 1.0000x reference)
import functools

import jax
import jax.numpy as jnp
from jax import lax
from jax.experimental import pallas as pl
from jax.experimental.pallas import tpu as pltpu
from jax.experimental.pallas import tpu_sc as plsc

_N_CHANNELS = 26
_EMBED_DIM = 32


@functools.lru_cache(maxsize=None)
def _make_gather(n_rows, dim):
    info = plsc.get_sparse_core_info()
    nc, ns, lanes = info.num_cores, info.num_subcores, info.num_lanes
    nw = nc * ns
    rows_per_w = n_rows // nw
    ch_rows = 1664
    n_chunks = rows_per_w // ch_rows
    n_streams = ch_rows // 128
    n_groups = ch_rows // lanes

    mesh = plsc.VectorSubcoreMesh(core_axis_name="c", subcore_axis_name="s")

    @functools.partial(
        pl.kernel,
        mesh=mesh,
        out_type=jax.ShapeDtypeStruct((n_rows, dim), jnp.float32),
        scratch_types=[
            pltpu.VMEM((ch_rows,), jnp.int32),
            pltpu.VMEM((ch_rows, dim), jnp.float32),
            pltpu.VMEM((32,), jnp.int32),
            pltpu.SemaphoreType.DMA,
        ],
    )
    def gather_kernel(idx_hbm, table_hbm, offs_hbm, out_hbm,
                      idx_v, rows_v, offs_v, sem):
        wid = lax.axis_index("s") * nc + lax.axis_index("c")
        pltpu.sync_copy(offs_hbm, offs_v)
        base_w = wid * rows_per_w

        def chunk_body(k, _):
            base = base_w + k * ch_rows
            pltpu.sync_copy(idx_hbm.at[pl.ds(base, ch_rows)], idx_v)

            def add_offsets(t, _):
                p0 = base + t * lanes
                chan = lax.rem(p0 + lax.iota(jnp.int32, lanes), _N_CHANNELS)
                off = plsc.load_gather(offs_v, [chan])
                idx_v[pl.ds(t * lanes, lanes)] = (
                    idx_v[pl.ds(t * lanes, lanes)] + off)
                return 0

            lax.fori_loop(0, n_groups, add_offsets, 0)

            copies = []
            for j in range(n_streams):
                copies.append(pltpu.async_copy(
                    table_hbm.at[idx_v.at[pl.ds(j * 128, 128)]],
                    rows_v.at[pl.ds(j * 128, 128)],
                    sem))
            for c in copies:
                c.wait()
            pltpu.sync_copy(rows_v, out_hbm.at[pl.ds(base, ch_rows)])
            return 0

        lax.fori_loop(0, n_chunks, chunk_body, 0)

    return gather_kernel


def kernel(inputs, table, offsets):
    b, c = inputs.shape
    _, d = table.shape
    idx_flat = inputs.astype(jnp.int32).reshape(-1)
    offs = jnp.pad(offsets.astype(jnp.int32), (0, 32 - c))
    out = _make_gather(b * c, d)(idx_flat, table, offs)
    return out.reshape(b, c * d)


# speedup vs baseline: 1.4622x; 1.4622x over previous
import functools

import jax
import jax.numpy as jnp
from jax import lax
from jax.experimental import pallas as pl
from jax.experimental.pallas import tpu as pltpu
from jax.experimental.pallas import tpu_sc as plsc

_N_CHANNELS = 26
_EMBED_DIM = 32


_QUARTER = 655360
_TBLK = 5120


def _transpose_body(x0, x1, x2, x3, out_ref):
    out_ref[...] = jnp.concatenate(
        [x0[...].T, x1[...].T, x2[...].T, x3[...].T], axis=1)


@functools.lru_cache(maxsize=None)
def _make_transpose(v, d):
    n_blocks = _QUARTER // _TBLK
    quarter_blocks = _QUARTER // _TBLK
    max_blk = pl.cdiv(v, _TBLK) - 1

    def spec(a):
        return pl.BlockSpec(
            (d, _TBLK),
            lambda i, a=a: (0, jnp.minimum(a * quarter_blocks + i, max_blk)))

    grid_spec = pl.GridSpec(
        grid=(n_blocks,),
        in_specs=[spec(0), spec(1), spec(2), spec(3)],
        out_specs=pl.BlockSpec((_TBLK, 4 * d), lambda i: (i, 0)),
    )
    return pl.pallas_call(
        _transpose_body,
        grid_spec=grid_spec,
        out_shape=jax.ShapeDtypeStruct((_QUARTER, 4 * d), jnp.float32),
    )


@functools.lru_cache(maxsize=None)
def _make_gather(n_rows, dim):
    info = plsc.get_sparse_core_info()
    nc, ns, lanes = info.num_cores, info.num_subcores, info.num_lanes
    nw = nc * ns
    rows_per_w = n_rows // nw
    ch_rows = 1664
    n_chunks = rows_per_w // ch_rows
    n_streams = ch_rows // 128
    n_groups = ch_rows // lanes

    mesh = plsc.VectorSubcoreMesh(core_axis_name="c", subcore_axis_name="s")

    @functools.partial(
        pl.kernel,
        mesh=mesh,
        out_type=jax.ShapeDtypeStruct((n_rows, dim), jnp.float32),
        compiler_params=pltpu.CompilerParams(use_tc_tiling_on_sc=False),
        scratch_types=[
            pltpu.VMEM((ch_rows,), jnp.int32),
            pltpu.VMEM((ch_rows, dim), jnp.float32),
            pltpu.VMEM((64,), jnp.int32),
            pltpu.SemaphoreType.DMA,
        ],
    )
    def gather_kernel(idx_hbm, table_hbm, offs_hbm, out_hbm,
                      idx_v, rows_v, offs_v, sem):
        wid = lax.axis_index("s") * nc + lax.axis_index("c")
        pltpu.sync_copy(offs_hbm, offs_v)
        base_w = wid * rows_per_w

        def chunk_body(k, _):
            base = base_w + k * ch_rows
            pltpu.sync_copy(idx_hbm.at[pl.ds(base, ch_rows)], idx_v)

            def add_offsets(t, _):
                p0 = base + t * lanes
                r = lax.rem(p0, _N_CHANNELS)
                off = offs_v[pl.ds(r, lanes)]
                full = idx_v[pl.ds(t * lanes, lanes)] + off
                q = lax.rem(full, _QUARTER)
                a = lax.div(full, _QUARTER)
                idx_v[pl.ds(t * lanes, lanes)] = q * 4 + a
                return 0

            lax.fori_loop(0, n_groups, add_offsets, 0)

            copies = []
            for j in range(n_streams):
                copies.append(pltpu.async_copy(
                    table_hbm.at[idx_v.at[pl.ds(j * 128, 128)]],
                    rows_v.at[pl.ds(j * 128, 128)],
                    sem))
            for c in copies:
                c.wait()
            pltpu.sync_copy(rows_v, out_hbm.at[pl.ds(base, ch_rows)])
            return 0

        lax.fori_loop(0, n_chunks, chunk_body, 0)

    return gather_kernel


def kernel(inputs, table, offsets):
    b, c = inputs.shape
    v, d = table.shape
    idx_flat = inputs.astype(jnp.int32).reshape(-1)
    offs = jnp.tile(offsets.astype(jnp.int32), 3)[:64]
    tt = table.T
    table_lin = _make_transpose(v, d)(tt, tt, tt, tt).reshape(4 * _QUARTER, d)
    out = _make_gather(b * c, d)(idx_flat, table_lin, offs)
    return out.reshape(b, c * d)
